# baseline (speedup 1.0000x reference)
import functools
import math

import jax
import jax.numpy as jnp
from jax import lax
from jax.experimental import pallas as pl
from jax.experimental.pallas import tpu as pltpu
from jax.experimental.pallas import tpu_sc as plsc

N = 10000
DIM = 128
NG = 32
MAXN = 625
SEEDS1 = 75
E = 320000

NP = 10752
RB = NP // 8
NW = 32
CH = 80
CHK = 160
EP = NW * CH * 128
RPT = NP // 16
NB = 1
IB = 16
NEG = -1e9
ISQ = 1.0 / math.sqrt(128.0)


def _cnt_ranges(sid, fn):
    @pl.when(sid < 4)
    def _():
        fn(pl.ds(sid * 768, 768))

    @pl.when(sid >= 4)
    def _():
        fn(pl.ds(3072 + (sid - 4) * 640, 640))


def _gather_scatter_loop(feat, srcp, dstp, base, nch, acc_sp, bufs, sems,
                         idx_s, idx_d, sem_is, sem_id,
                         cnt_sp=None, ones_v=None):
    pltpu.sync_copy(srcp.at[pl.ds(base, nch)], idx_s)
    pltpu.sync_copy(dstp.at[pl.ds(base, nch)], idx_d)

    def body(j, carry):
        pltpu.async_copy(feat.at[idx_s.at[j]], bufs[0], sems[0]).wait()
        pltpu.sync_copy(bufs[0], acc_sp.at[idx_d.at[j]], add=True)
        if cnt_sp is not None:
            pltpu.sync_copy(ones_v, cnt_sp.at[idx_d.at[j]], add=True)
        return carry

    lax.fori_loop(0, nch, body, 0)


def _sage_body_factory(with_cnt):
    def body(*refs):
        if with_cnt:
            (feat, init0, init1, zvec, ones_hbm, srcp, dstp,
             agg_out, cnt_out, idx_s, idx_d) = refs[:11]
            bufs = refs[11:11 + NB]
            ones_v, acc_sp, cnt_sp = refs[11 + NB:14 + NB]
            sems = refs[14 + NB:]
        else:
            (feat, init0, init1, srcp, dstp,
             agg_out, idx_s, idx_d) = refs[:8]
            bufs = refs[8:8 + NB]
            acc_sp = refs[8 + NB]
            sems = refs[9 + NB:]
        cid = lax.axis_index("c")
        sid = lax.axis_index("s")
        wid = sid * 2 + cid
        rng = pl.ds(sid * RPT, RPT)

        @pl.when(cid == 0)
        def _():
            pltpu.sync_copy(init0.at[rng], acc_sp.at[rng])

        @pl.when(cid == 1)
        def _():
            pltpu.sync_copy(init1.at[rng], acc_sp.at[rng])

        if with_cnt:
            _cnt_ranges(sid, lambda r: pltpu.sync_copy(zvec.at[r],
                                                       cnt_sp.at[r]))
            pltpu.sync_copy(ones_hbm, ones_v)
        plsc.subcore_barrier()
        _gather_scatter_loop(feat, srcp, dstp, wid * CH, CH, acc_sp,
                             list(bufs), list(sems[:NB]), idx_s, idx_d,
                             sems[NB], sems[NB + 1],
                             cnt_sp=cnt_sp if with_cnt else None,
                             ones_v=ones_v if with_cnt else None)
        plsc.subcore_barrier()
        pltpu.sync_copy(acc_sp.at[rng], agg_out.at[cid].at[rng])
        if with_cnt:
            _cnt_ranges(sid, lambda r: pltpu.sync_copy(
                cnt_sp.at[r], cnt_out.at[cid].at[r]))
    return body


def _kv_body(*refs):
    (featk, featv, initk, initv, srcp, dstp,
     agg_out, idx_s, idx_d) = refs[:9]
    bufs = list(refs[9:9 + NB])
    acc_sp = refs[9 + NB]
    sems = list(refs[10 + NB:])
    cid = lax.axis_index("c")
    sid = lax.axis_index("s")
    rng = pl.ds(sid * RPT, RPT)

    @pl.when(cid == 0)
    def _():
        pltpu.sync_copy(initk.at[rng], acc_sp.at[rng])

    @pl.when(cid == 1)
    def _():
        pltpu.sync_copy(initv.at[rng], acc_sp.at[rng])

    plsc.subcore_barrier()

    @pl.when(cid == 0)
    def _():
        _gather_scatter_loop(featk, srcp, dstp, sid * CHK, CHK, acc_sp,
                             bufs, sems[:NB], idx_s, idx_d,
                             sems[NB], sems[NB + 1])

    @pl.when(cid == 1)
    def _():
        _gather_scatter_loop(featv, srcp, dstp, sid * CHK, CHK, acc_sp,
                             bufs, sems[:NB], idx_s, idx_d,
                             sems[NB], sems[NB + 1])

    plsc.subcore_barrier()
    pltpu.sync_copy(acc_sp.at[rng], agg_out.at[cid].at[rng])


def _sc_scratch(with_cnt):
    scratch = [
        pltpu.VMEM((CH, 128), jnp.int32),
        pltpu.VMEM((CH, 128), jnp.int32),
    ]
    scratch += [pltpu.VMEM((128, DIM), jnp.float32) for _ in range(NB)]
    if with_cnt:
        scratch.append(pltpu.VMEM((128,), jnp.float32))
    scratch.append(pltpu.VMEM_SHARED((NP, DIM), jnp.float32))
    if with_cnt:
        scratch.append(pltpu.VMEM_SHARED((NP,), jnp.float32))
    scratch += [pltpu.SemaphoreType.DMA for _ in range(NB + 2)]
    return scratch


def _mesh():
    return plsc.VectorSubcoreMesh(core_axis_name="c", subcore_axis_name="s")


def _sc_pass(feat, init0, init1, srcp, dstp, zvec=None, ones=None):
    with_cnt = zvec is not None
    outs = [jax.ShapeDtypeStruct((2, NP, DIM), jnp.float32)]
    if with_cnt:
        outs.append(jax.ShapeDtypeStruct((2, NP), jnp.float32))
    k = pl.kernel(
        _sage_body_factory(with_cnt),
        mesh=_mesh(),
        out_type=outs if with_cnt else outs[0],
        scratch_types=_sc_scratch(with_cnt),
    )
    if with_cnt:
        return k(feat, init0, init1, zvec, ones, srcp, dstp)
    return k(feat, init0, init1, srcp, dstp)


def _sc_kv_pass(featk, featv, initk, initv, srcp, dstp):
    k = pl.kernel(
        _kv_body,
        mesh=_mesh(),
        out_type=jax.ShapeDtypeStruct((2, NP, DIM), jnp.float32),
        scratch_types=_sc_scratch(False),
    )
    return k(featk, featv, initk, initv, srcp, dstp)


def _full(shape):
    return pl.BlockSpec(shape, lambda i: tuple(0 for _ in shape))


def _rows(shape):
    if len(shape) == 2:
        return pl.BlockSpec(shape, lambda i: (i, 0))
    return pl.BlockSpec(shape, lambda i: (0, i, 0))


def _stage_a(xp, W, b):
    def body(x_ref, w_ref, b_ref, o_ref):
        o_ref[...] = jnp.maximum(
            jnp.dot(x_ref[...], w_ref[...],
                    preferred_element_type=jnp.float32) + b_ref[...], 0.0)
    return pl.pallas_call(
        body,
        grid=(8,),
        in_specs=[_rows((RB, DIM)), _full((DIM, DIM)), _full((1, DIM))],
        out_specs=_rows((RB, DIM)),
        out_shape=jax.ShapeDtypeStruct((NP, DIM), jnp.float32),
    )(xp, W, b)


def _prep(cntp):
    def body(c_ref, rc_ref, di_ref):
        c = c_ref[0] + c_ref[1]
        rc_ref[...] = 1.0 / jnp.maximum(c, 1.0)
        di_ref[...] = lax.rsqrt(c + 1.0)
    return pl.pallas_call(
        body,
        grid=(1,),
        in_specs=[_full((2, NP, 1))],
        out_specs=[_full((NP, 1)), _full((NP, 1))],
        out_shape=[jax.ShapeDtypeStruct((NP, 1), jnp.float32),
                   jax.ShapeDtypeStruct((NP, 1), jnp.float32)],
    )(cntp)


def _stage_b(aggp, h, rc, W_l, b_l, W_r):
    def body(a_ref, h_ref, rc_ref, wl_ref, bl_ref, wr_ref, o_ref):
        mean = (a_ref[0] + a_ref[1]) * rc_ref[...]
        o_ref[...] = jnp.maximum(
            jnp.dot(mean, wl_ref[...], preferred_element_type=jnp.float32)
            + bl_ref[...]
            + jnp.dot(h_ref[...], wr_ref[...],
                      preferred_element_type=jnp.float32), 0.0)
    return pl.pallas_call(
        body,
        grid=(8,),
        in_specs=[_rows((2, RB, DIM)), _rows((RB, DIM)), _rows((RB, 1)),
                  _full((DIM, DIM)), _full((1, DIM)), _full((DIM, DIM))],
        out_specs=_rows((RB, DIM)),
        out_shape=jax.ShapeDtypeStruct((NP, DIM), jnp.float32),
    )(aggp, h, rc, W_l, b_l, W_r)


def _stage_c(aggp, h, rc, dinv, W_l, b_l, W_r, lin1W, lin1b, Wk, Wv):
    def body(a_ref, h_ref, rc_ref, di_ref, wl_ref, bl_ref, wr_ref,
             l1w_ref, l1b_ref, wk_ref, wv_ref, hk_ref, hv_ref):
        mean = (a_ref[0] + a_ref[1]) * rc_ref[...]
        h3 = jnp.maximum(
            jnp.dot(mean, wl_ref[...], preferred_element_type=jnp.float32)
            + bl_ref[...]
            + jnp.dot(h_ref[...], wr_ref[...],
                      preferred_element_type=jnp.float32), 0.0)
        xpv = jnp.dot(h3, l1w_ref[...],
                      preferred_element_type=jnp.float32) + l1b_ref[...]
        di = di_ref[...]
        hk_ref[...] = jnp.dot(xpv, wk_ref[...],
                              preferred_element_type=jnp.float32) * di
        hv_ref[...] = jnp.dot(xpv, wv_ref[...],
                              preferred_element_type=jnp.float32) * di
    return pl.pallas_call(
        body,
        grid=(8,),
        in_specs=[_rows((2, RB, DIM)), _rows((RB, DIM)), _rows((RB, 1)),
                  _rows((RB, 1)), _full((DIM, DIM)), _full((1, DIM)),
                  _full((DIM, DIM)), _full((DIM, DIM)), _full((1, DIM)),
                  _full((DIM, DIM)), _full((DIM, DIM))],
        out_specs=[_rows((RB, DIM)), _rows((RB, DIM))],
        out_shape=[jax.ShapeDtypeStruct((NP, DIM), jnp.float32),
                   jax.ShapeDtypeStruct((NP, DIM), jnp.float32)],
    )(aggp, h, rc, dinv, W_l, b_l, W_r, lin1W, lin1b, Wk, Wv)


def _softmax_rows(s):
    m = jnp.max(s, axis=1, keepdims=True)
    e = jnp.exp(s - m)
    return e / jnp.sum(e, axis=1, keepdims=True)


def _heads_attn(Qp, K, V, maskrow=None):
    outs = []
    for h in range(4):
        qh = Qp[:, h * 32:(h + 1) * 32]
        kh = K[:, h * 32:(h + 1) * 32]
        vh = V[:, h * 32:(h + 1) * 32]
        s = lax.dot_general(qh, kh, (((1,), (1,)), ((), ())),
                            preferred_element_type=jnp.float32) * ISQ
        if maskrow is not None:
            s = s + maskrow
        a = _softmax_rows(s)
        outs.append(jnp.dot(a, vh, preferred_element_type=jnp.float32))
    return Qp + jnp.concatenate(outs, axis=1)


def _attention(kn0, kn1, vn0, vn1, dinv, batch2d, wts):
    (S1, Wq1, bq1, Wo1, bo1, bk1, bv1,
     Wq2, bq2, Wk2, bk2, Wv2, bv2, Wo2, bo2,
     S3, Wq3, bq3, Wk3, bk3, Wv3, bv3, Wo3, bo3,
     l2W, l2b) = wts

    def body(k0_ref, k1_ref, v0_ref, v1_ref, di_ref, b_ref,
             s1_ref, wq1_ref, bq1_ref, wo1_ref, bo1_ref, bk1_ref, bv1_ref,
             wq2_ref, bq2_ref, wk2_ref, bk2_ref, wv2_ref, bv2_ref,
             wo2_ref, bo2_ref,
             s3_ref, wq3_ref, bq3_ref, wk3_ref, bk3_ref, wv3_ref, bv3_ref,
             wo3_ref, bo3_ref, l2w_ref, l2b_ref, o_ref):
        g = pl.program_id(0)
        b2 = b_ref[...]
        cntg = jnp.sum((b2 == g).astype(jnp.int32))
        startg = jnp.sum((b2 < g).astype(jnp.int32))
        sl = pl.ds(startg, 640)
        di = di_ref[sl, :]
        K = di * (k0_ref[sl, :] + k1_ref[sl, :]) + bk1_ref[...]
        V = di * (v0_ref[sl, :] + v1_ref[sl, :]) + bv1_ref[...]
        kmax = jnp.minimum(cntg, MAXN)
        vcol = lax.broadcasted_iota(jnp.int32, (640, 1), 0) < kmax
        vrow = lax.broadcasted_iota(jnp.int32, (1, 640), 1) < kmax
        V = jnp.where(vcol, V, 0.0)
        maskrow = jnp.where(vrow, 0.0, NEG)

        Qp = jnp.dot(s1_ref[...], wq1_ref[...],
                     preferred_element_type=jnp.float32) + bq1_ref[...]
        X = _heads_attn(Qp, K, V, maskrow)
        X = X + jnp.maximum(
            jnp.dot(X, wo1_ref[...], preferred_element_type=jnp.float32)
            + bo1_ref[...], 0.0)

        Q2 = jnp.dot(X, wq2_ref[...],
                     preferred_element_type=jnp.float32) + bq2_ref[...]
        K2 = jnp.dot(X, wk2_ref[...],
                     preferred_element_type=jnp.float32) + bk2_ref[...]
        V2 = jnp.dot(X, wv2_ref[...],
                     preferred_element_type=jnp.float32) + bv2_ref[...]
        X2 = _heads_attn(Q2, K2, V2)
        X2 = X2 + jnp.maximum(
            jnp.dot(X2, wo2_ref[...], preferred_element_type=jnp.float32)
            + bo2_ref[...], 0.0)

        Q3 = jnp.dot(s3_ref[...], wq3_ref[...],
                     preferred_element_type=jnp.float32) + bq3_ref[...]
        K3 = jnp.dot(X2, wk3_ref[...],
                     preferred_element_type=jnp.float32) + bk3_ref[...]
        V3 = jnp.dot(X2, wv3_ref[...],
                     preferred_element_type=jnp.float32) + bv3_ref[...]
        X3 = _heads_attn(Q3, K3, V3)
        X3 = X3 + jnp.maximum(
            jnp.dot(X3, wo3_ref[...], preferred_element_type=jnp.float32)
            + bo3_ref[...], 0.0)
        y = jnp.dot(X3, l2w_ref[...],
                    preferred_element_type=jnp.float32) + l2b_ref[...]
        o_ref[...] = jnp.broadcast_to(y[None], (1, 8, DIM))

    nspecs = [_full((NP, DIM))] * 4 + [_full((NP, 1)), _full((84, 128))]
    wspecs = [_full(w.shape) for w in wts]
    return pl.pallas_call(
        body,
        grid=(NG,),
        in_specs=nspecs + wspecs,
        out_specs=pl.BlockSpec((1, 8, DIM), lambda i: (i, 0, 0)),
        out_shape=jax.ShapeDtypeStruct((NG, 8, DIM), jnp.float32),
    )(kn0, kn1, vn0, vn1, dinv, batch2d, *wts)


def kernel(x, edge_index, batch, params):
    p = params
    xpad = jnp.pad(x, ((0, NP - N), (0, 0)))
    srcp = jnp.pad(edge_index[0], (0, EP - E),
                   constant_values=N).reshape(EP // 128, 128)
    dstp = jnp.pad(edge_index[1], (0, EP - E),
                   constant_values=N).reshape(EP // 128, 128)
    batch2d = jnp.pad(batch, (0, NP - N),
                      constant_values=NG).reshape(84, 128)
    z128 = jnp.zeros((NP, DIM), jnp.float32)
    zvec = jnp.zeros((NP,), jnp.float32)
    ones = jnp.ones((128,), jnp.float32)

    def r2(b):
        return b.reshape(1, DIM)

    h0 = _stage_a(xpad, p["W_in"], r2(p["b_in"]))
    agg0, cntp = _sc_pass(h0, z128, z128, srcp, dstp, zvec=zvec, ones=ones)
    rc, dinv = _prep(cntp.reshape(2, NP, 1))
    h1 = _stage_b(agg0, h0, rc, p["W_l0"], r2(p["b_l0"]), p["W_r0"])
    agg1 = _sc_pass(h1, z128, z128, srcp, dstp)
    h2 = _stage_b(agg1, h1, rc, p["W_l1"], r2(p["b_l1"]), p["W_r1"])
    agg2 = _sc_pass(h2, z128, z128, srcp, dstp)
    hk, hv = _stage_c(agg2, h2, rc, dinv, p["W_l2"], r2(p["b_l2"]),
                      p["W_r2"], p["lin1_W"], r2(p["lin1_b"]),
                      p["p1"]["Wk"], p["p1"]["Wv"])
    kvk = _sc_pass(hk, hk, z128, srcp, dstp)
    kvv = _sc_pass(hv, hv, z128, srcp, dstp)
    kv = (kvk, kvv)

    m1, m2, m3 = p["p1"], p["p2"], p["p3"]
    wts = (p["S1"][0], m1["Wq"], r2(m1["bq"]), m1["Wo"], r2(m1["bo"]),
           r2(m1["bk"]), r2(m1["bv"]),
           m2["Wq"], r2(m2["bq"]), m2["Wk"], r2(m2["bk"]),
           m2["Wv"], r2(m2["bv"]), m2["Wo"], r2(m2["bo"]),
           p["S3"][0], m3["Wq"], r2(m3["bq"]), m3["Wk"], r2(m3["bk"]),
           m3["Wv"], r2(m3["bv"]), m3["Wo"], r2(m3["bo"]),
           p["lin2_W"], p["lin2_b"].reshape(1, 1))
    y = _attention(kv[0][0], kv[0][1], kv[1][0], kv[1][1],
                   dinv, batch2d, wts)
    return y[:, 0, 0]


# speedup vs baseline: 1.0578x; 1.0578x over previous
import functools
import math

import jax
import jax.numpy as jnp
from jax import lax
from jax.experimental import pallas as pl
from jax.experimental.pallas import tpu as pltpu
from jax.experimental.pallas import tpu_sc as plsc

N = 10000
DIM = 128
NG = 32
MAXN = 625
SEEDS1 = 75
E = 320000

NP = 10752
RB = NP // 8
NW = 32
CH = 80
CHK = 160
EP = NW * CH * 128
RPT = NP // 16
NB = 1
IB = 16
NEG = -1e9
ISQ = 1.0 / math.sqrt(128.0)


def _cnt_ranges(sid, fn):
    @pl.when(sid < 4)
    def _():
        fn(pl.ds(sid * 768, 768))

    @pl.when(sid >= 4)
    def _():
        fn(pl.ds(3072 + (sid - 4) * 640, 640))


def _gather_scatter_loop(feat, srcp, dstp, base, nch, acc_sp, bufs, sems,
                         idx_s, idx_d, sem_is, sem_id,
                         cnt_sp=None, ones_v=None):
    pltpu.sync_copy(srcp.at[base], idx_s)
    pltpu.sync_copy(dstp.at[base], idx_d)

    def body(j, carry):
        pltpu.async_copy(feat.at[idx_s.at[j]], bufs[0], sems[0]).wait()
        pltpu.sync_copy(bufs[0], acc_sp.at[idx_d.at[j]], add=True)
        if cnt_sp is not None:
            pltpu.sync_copy(ones_v, cnt_sp.at[idx_d.at[j]], add=True)
        return carry

    lax.fori_loop(0, nch, body, 0)


def _sage_body_factory(with_cnt):
    def body(*refs):
        if with_cnt:
            (feat, init0, init1, zvec, ones_hbm, srcp, dstp,
             agg_out, cnt_out, idx_s, idx_d) = refs[:11]
            bufs = refs[11:11 + NB]
            ones_v, acc_sp, cnt_sp = refs[11 + NB:14 + NB]
            sems = refs[14 + NB:]
        else:
            (feat, init0, init1, srcp, dstp,
             agg_out, idx_s, idx_d) = refs[:8]
            bufs = refs[8:8 + NB]
            acc_sp = refs[8 + NB]
            sems = refs[9 + NB:]
        cid = lax.axis_index("c")
        sid = lax.axis_index("s")
        wid = sid * 2 + cid
        rng = pl.ds(sid * RPT, RPT)

        @pl.when(cid == 0)
        def _():
            pltpu.sync_copy(init0.at[rng], acc_sp.at[rng])

        @pl.when(cid == 1)
        def _():
            pltpu.sync_copy(init1.at[rng], acc_sp.at[rng])

        if with_cnt:
            _cnt_ranges(sid, lambda r: pltpu.sync_copy(zvec.at[r],
                                                       cnt_sp.at[r]))
            pltpu.sync_copy(ones_hbm, ones_v)
        plsc.subcore_barrier()
        _gather_scatter_loop(feat, srcp, dstp, wid, CH, acc_sp,
                             list(bufs), list(sems[:NB]), idx_s, idx_d,
                             sems[NB], sems[NB + 1],
                             cnt_sp=cnt_sp if with_cnt else None,
                             ones_v=ones_v if with_cnt else None)
        plsc.subcore_barrier()
        pltpu.sync_copy(acc_sp.at[rng], agg_out.at[cid].at[rng])
        if with_cnt:
            _cnt_ranges(sid, lambda r: pltpu.sync_copy(
                cnt_sp.at[r], cnt_out.at[cid].at[r]))
    return body


def _kv_body(*refs):
    (featk, featv, initk, initv, srcp, dstp,
     agg_out, idx_s, idx_d) = refs[:9]
    bufs = list(refs[9:9 + NB])
    acc_sp = refs[9 + NB]
    sems = list(refs[10 + NB:])
    cid = lax.axis_index("c")
    sid = lax.axis_index("s")
    rng = pl.ds(sid * RPT, RPT)

    @pl.when(cid == 0)
    def _():
        pltpu.sync_copy(initk.at[rng], acc_sp.at[rng])

    @pl.when(cid == 1)
    def _():
        pltpu.sync_copy(initv.at[rng], acc_sp.at[rng])

    plsc.subcore_barrier()

    @pl.when(cid == 0)
    def _():
        _gather_scatter_loop(featk, srcp, dstp, sid * CHK, CHK, acc_sp,
                             bufs, sems[:NB], idx_s, idx_d,
                             sems[NB], sems[NB + 1])

    @pl.when(cid == 1)
    def _():
        _gather_scatter_loop(featv, srcp, dstp, sid * CHK, CHK, acc_sp,
                             bufs, sems[:NB], idx_s, idx_d,
                             sems[NB], sems[NB + 1])

    plsc.subcore_barrier()
    pltpu.sync_copy(acc_sp.at[rng], agg_out.at[cid].at[rng])


def _sc_scratch(with_cnt):
    scratch = [
        pltpu.VMEM((CH, 128), jnp.int32),
        pltpu.VMEM((CH, 128), jnp.int32),
    ]
    scratch += [pltpu.VMEM((128, DIM), jnp.float32) for _ in range(NB)]
    if with_cnt:
        scratch.append(pltpu.VMEM((128,), jnp.float32))
    scratch.append(pltpu.VMEM_SHARED((NP, DIM), jnp.float32))
    if with_cnt:
        scratch.append(pltpu.VMEM_SHARED((NP,), jnp.float32))
    scratch += [pltpu.SemaphoreType.DMA for _ in range(NB + 2)]
    return scratch


def _mesh():
    return plsc.VectorSubcoreMesh(core_axis_name="c", subcore_axis_name="s")


def _sc_pass(feat, init0, init1, srcp, dstp, zvec=None, ones=None):
    with_cnt = zvec is not None
    outs = [jax.ShapeDtypeStruct((2, NP, DIM), jnp.float32)]
    if with_cnt:
        outs.append(jax.ShapeDtypeStruct((2, NP), jnp.float32))
    k = pl.kernel(
        _sage_body_factory(with_cnt),
        mesh=_mesh(),
        out_type=outs if with_cnt else outs[0],
        scratch_types=_sc_scratch(with_cnt),
    )
    if with_cnt:
        return k(feat, init0, init1, zvec, ones, srcp, dstp)
    return k(feat, init0, init1, srcp, dstp)


def _sc_kv_pass(featk, featv, initk, initv, srcp, dstp):
    k = pl.kernel(
        _kv_body,
        mesh=_mesh(),
        out_type=jax.ShapeDtypeStruct((2, NP, DIM), jnp.float32),
        scratch_types=_sc_scratch(False),
    )
    return k(featk, featv, initk, initv, srcp, dstp)


def _full(shape):
    return pl.BlockSpec(shape, lambda i: tuple(0 for _ in shape))


def _rows(shape):
    if len(shape) == 2:
        return pl.BlockSpec(shape, lambda i: (i, 0))
    return pl.BlockSpec(shape, lambda i: (0, i, 0))


def _stage_a(xp, W, b):
    def body(x_ref, w_ref, b_ref, o_ref):
        o_ref[...] = jnp.maximum(
            jnp.dot(x_ref[...], w_ref[...],
                    preferred_element_type=jnp.float32) + b_ref[...], 0.0)
    return pl.pallas_call(
        body,
        grid=(8,),
        in_specs=[_rows((RB, DIM)), _full((DIM, DIM)), _full((1, DIM))],
        out_specs=_rows((RB, DIM)),
        out_shape=jax.ShapeDtypeStruct((NP, DIM), jnp.float32),
    )(xp, W, b)


def _prep(cntp):
    def body(c_ref, rc_ref, di_ref):
        c = c_ref[0] + c_ref[1]
        rc_ref[...] = 1.0 / jnp.maximum(c, 1.0)
        di_ref[...] = lax.rsqrt(c + 1.0)
    return pl.pallas_call(
        body,
        grid=(1,),
        in_specs=[_full((2, NP, 1))],
        out_specs=[_full((NP, 1)), _full((NP, 1))],
        out_shape=[jax.ShapeDtypeStruct((NP, 1), jnp.float32),
                   jax.ShapeDtypeStruct((NP, 1), jnp.float32)],
    )(cntp)


def _stage_b(aggp, h, rc, W_l, b_l, W_r):
    def body(a_ref, h_ref, rc_ref, wl_ref, bl_ref, wr_ref, o_ref):
        mean = (a_ref[0] + a_ref[1]) * rc_ref[...]
        o_ref[...] = jnp.maximum(
            jnp.dot(mean, wl_ref[...], preferred_element_type=jnp.float32)
            + bl_ref[...]
            + jnp.dot(h_ref[...], wr_ref[...],
                      preferred_element_type=jnp.float32), 0.0)
    return pl.pallas_call(
        body,
        grid=(8,),
        in_specs=[_rows((2, RB, DIM)), _rows((RB, DIM)), _rows((RB, 1)),
                  _full((DIM, DIM)), _full((1, DIM)), _full((DIM, DIM))],
        out_specs=_rows((RB, DIM)),
        out_shape=jax.ShapeDtypeStruct((NP, DIM), jnp.float32),
    )(aggp, h, rc, W_l, b_l, W_r)


def _stage_c(aggp, h, rc, dinv, W_l, b_l, W_r, lin1W, lin1b, Wk, Wv):
    def body(a_ref, h_ref, rc_ref, di_ref, wl_ref, bl_ref, wr_ref,
             l1w_ref, l1b_ref, wk_ref, wv_ref, hk_ref, hv_ref):
        mean = (a_ref[0] + a_ref[1]) * rc_ref[...]
        h3 = jnp.maximum(
            jnp.dot(mean, wl_ref[...], preferred_element_type=jnp.float32)
            + bl_ref[...]
            + jnp.dot(h_ref[...], wr_ref[...],
                      preferred_element_type=jnp.float32), 0.0)
        xpv = jnp.dot(h3, l1w_ref[...],
                      preferred_element_type=jnp.float32) + l1b_ref[...]
        di = di_ref[...]
        hk_ref[...] = jnp.dot(xpv, wk_ref[...],
                              preferred_element_type=jnp.float32) * di
        hv_ref[...] = jnp.dot(xpv, wv_ref[...],
                              preferred_element_type=jnp.float32) * di
    return pl.pallas_call(
        body,
        grid=(8,),
        in_specs=[_rows((2, RB, DIM)), _rows((RB, DIM)), _rows((RB, 1)),
                  _rows((RB, 1)), _full((DIM, DIM)), _full((1, DIM)),
                  _full((DIM, DIM)), _full((DIM, DIM)), _full((1, DIM)),
                  _full((DIM, DIM)), _full((DIM, DIM))],
        out_specs=[_rows((RB, DIM)), _rows((RB, DIM))],
        out_shape=[jax.ShapeDtypeStruct((NP, DIM), jnp.float32),
                   jax.ShapeDtypeStruct((NP, DIM), jnp.float32)],
    )(aggp, h, rc, dinv, W_l, b_l, W_r, lin1W, lin1b, Wk, Wv)


def _softmax_rows(s):
    m = jnp.max(s, axis=1, keepdims=True)
    e = jnp.exp(s - m)
    return e / jnp.sum(e, axis=1, keepdims=True)


def _heads_attn(Qp, K, V, maskrow=None):
    outs = []
    for h in range(4):
        qh = Qp[:, h * 32:(h + 1) * 32]
        kh = K[:, h * 32:(h + 1) * 32]
        vh = V[:, h * 32:(h + 1) * 32]
        s = lax.dot_general(qh, kh, (((1,), (1,)), ((), ())),
                            preferred_element_type=jnp.float32) * ISQ
        if maskrow is not None:
            s = s + maskrow
        a = _softmax_rows(s)
        outs.append(jnp.dot(a, vh, preferred_element_type=jnp.float32))
    return Qp + jnp.concatenate(outs, axis=1)


def _attention(kn0, kn1, vn0, vn1, dinv, batch2d, wts):
    (S1, Wq1, bq1, Wo1, bo1, bk1, bv1,
     Wq2, bq2, Wk2, bk2, Wv2, bv2, Wo2, bo2,
     S3, Wq3, bq3, Wk3, bk3, Wv3, bv3, Wo3, bo3,
     l2W, l2b) = wts

    def body(k0_ref, k1_ref, v0_ref, v1_ref, di_ref, b_ref,
             s1_ref, wq1_ref, bq1_ref, wo1_ref, bo1_ref, bk1_ref, bv1_ref,
             wq2_ref, bq2_ref, wk2_ref, bk2_ref, wv2_ref, bv2_ref,
             wo2_ref, bo2_ref,
             s3_ref, wq3_ref, bq3_ref, wk3_ref, bk3_ref, wv3_ref, bv3_ref,
             wo3_ref, bo3_ref, l2w_ref, l2b_ref, o_ref):
        g = pl.program_id(0)
        b2 = b_ref[...]
        cntg = jnp.sum((b2 == g).astype(jnp.int32))
        startg = jnp.sum((b2 < g).astype(jnp.int32))
        sl = pl.ds(startg, 640)
        di = di_ref[sl, :]
        K = di * (k0_ref[sl, :] + k1_ref[sl, :]) + bk1_ref[...]
        V = di * (v0_ref[sl, :] + v1_ref[sl, :]) + bv1_ref[...]
        kmax = jnp.minimum(cntg, MAXN)
        vcol = lax.broadcasted_iota(jnp.int32, (640, 1), 0) < kmax
        vrow = lax.broadcasted_iota(jnp.int32, (1, 640), 1) < kmax
        V = jnp.where(vcol, V, 0.0)
        maskrow = jnp.where(vrow, 0.0, NEG)

        Qp = jnp.dot(s1_ref[...], wq1_ref[...],
                     preferred_element_type=jnp.float32) + bq1_ref[...]
        X = _heads_attn(Qp, K, V, maskrow)
        X = X + jnp.maximum(
            jnp.dot(X, wo1_ref[...], preferred_element_type=jnp.float32)
            + bo1_ref[...], 0.0)

        Q2 = jnp.dot(X, wq2_ref[...],
                     preferred_element_type=jnp.float32) + bq2_ref[...]
        K2 = jnp.dot(X, wk2_ref[...],
                     preferred_element_type=jnp.float32) + bk2_ref[...]
        V2 = jnp.dot(X, wv2_ref[...],
                     preferred_element_type=jnp.float32) + bv2_ref[...]
        X2 = _heads_attn(Q2, K2, V2)
        X2 = X2 + jnp.maximum(
            jnp.dot(X2, wo2_ref[...], preferred_element_type=jnp.float32)
            + bo2_ref[...], 0.0)

        Q3 = jnp.dot(s3_ref[...], wq3_ref[...],
                     preferred_element_type=jnp.float32) + bq3_ref[...]
        K3 = jnp.dot(X2, wk3_ref[...],
                     preferred_element_type=jnp.float32) + bk3_ref[...]
        V3 = jnp.dot(X2, wv3_ref[...],
                     preferred_element_type=jnp.float32) + bv3_ref[...]
        X3 = _heads_attn(Q3, K3, V3)
        X3 = X3 + jnp.maximum(
            jnp.dot(X3, wo3_ref[...], preferred_element_type=jnp.float32)
            + bo3_ref[...], 0.0)
        y = jnp.dot(X3, l2w_ref[...],
                    preferred_element_type=jnp.float32) + l2b_ref[...]
        o_ref[...] = jnp.broadcast_to(y[None], (1, 8, DIM))

    nspecs = [_full((NP, DIM))] * 4 + [_full((NP, 1)), _full((84, 128))]
    wspecs = [_full(w.shape) for w in wts]
    return pl.pallas_call(
        body,
        grid=(NG,),
        in_specs=nspecs + wspecs,
        out_specs=pl.BlockSpec((1, 8, DIM), lambda i: (i, 0, 0)),
        out_shape=jax.ShapeDtypeStruct((NG, 8, DIM), jnp.float32),
    )(kn0, kn1, vn0, vn1, dinv, batch2d, *wts)


def kernel(x, edge_index, batch, params):
    p = params
    xpad = jnp.pad(x, ((0, NP - N), (0, 0)))
    srcp = jnp.pad(edge_index[0], (0, EP - E),
                   constant_values=N).reshape(NW, CH, 128)
    dstp = jnp.pad(edge_index[1], (0, EP - E),
                   constant_values=N).reshape(NW, CH, 128)
    batch2d = jnp.pad(batch, (0, NP - N),
                      constant_values=NG).reshape(84, 128)
    z128 = jnp.zeros((NP, DIM), jnp.float32)
    zvec = jnp.zeros((NP,), jnp.float32)
    ones = jnp.ones((128,), jnp.float32)

    def r2(b):
        return b.reshape(1, DIM)

    h0 = _stage_a(xpad, p["W_in"], r2(p["b_in"]))
    agg0, cntp = _sc_pass(h0, z128, z128, srcp, dstp, zvec=zvec, ones=ones)
    rc, dinv = _prep(cntp.reshape(2, NP, 1))
    h1 = _stage_b(agg0, h0, rc, p["W_l0"], r2(p["b_l0"]), p["W_r0"])
    agg1 = _sc_pass(h1, z128, z128, srcp, dstp)
    h2 = _stage_b(agg1, h1, rc, p["W_l1"], r2(p["b_l1"]), p["W_r1"])
    agg2 = _sc_pass(h2, z128, z128, srcp, dstp)
    hk, hv = _stage_c(agg2, h2, rc, dinv, p["W_l2"], r2(p["b_l2"]),
                      p["W_r2"], p["lin1_W"], r2(p["lin1_b"]),
                      p["p1"]["Wk"], p["p1"]["Wv"])
    kvk = _sc_pass(hk, hk, z128, srcp, dstp)
    kvv = _sc_pass(hv, hv, z128, srcp, dstp)
    kv = (kvk, kvv)

    m1, m2, m3 = p["p1"], p["p2"], p["p3"]
    wts = (p["S1"][0], m1["Wq"], r2(m1["bq"]), m1["Wo"], r2(m1["bo"]),
           r2(m1["bk"]), r2(m1["bv"]),
           m2["Wq"], r2(m2["bq"]), m2["Wk"], r2(m2["bk"]),
           m2["Wv"], r2(m2["bv"]), m2["Wo"], r2(m2["bo"]),
           p["S3"][0], m3["Wq"], r2(m3["bq"]), m3["Wk"], r2(m3["bk"]),
           m3["Wv"], r2(m3["bv"]), m3["Wo"], r2(m3["bo"]),
           p["lin2_W"], p["lin2_b"].reshape(1, 1))
    y = _attention(kv[0][0], kv[0][1], kv[1][0], kv[1][1],
                   dinv, batch2d, wts)
    return y[:, 0, 0]


# speedup vs baseline: 1.5309x; 1.4473x over previous
import functools
import math

import jax
import jax.numpy as jnp
from jax import lax
from jax.experimental import pallas as pl
from jax.experimental.pallas import tpu as pltpu
from jax.experimental.pallas import tpu_sc as plsc

N = 10000
DIM = 128
NG = 32
MAXN = 625
SEEDS1 = 75
E = 320000

NP = 10752
RB = NP // 8
NW = 32
CH = 79
EP = NW * CH * 128
RPT = NP // 16
NEG = -1e9
ISQ = 1.0 / math.sqrt(128.0)


def _sc_body_factory(with_cnt):
    def body(*refs):
        if with_cnt:
            (feat, init0, init1, zvec, ones_hbm, srcp, dstp,
             agg_out, cnt_out,
             idx_s, idx_d, rows, ones_v, acc_sp, cnt_sp, sem) = refs
        else:
            (feat, init0, init1, srcp, dstp,
             agg_out,
             idx_s, idx_d, rows, acc_sp, sem) = refs
        cid = lax.axis_index("c")
        sid = lax.axis_index("s")
        wid = sid * 2 + cid
        rng = pl.ds(sid * RPT, RPT)

        def cnt_ranges(fn):
            @pl.when(sid < 4)
            def _():
                fn(pl.ds(sid * 768, 768))

            @pl.when(sid >= 4)
            def _():
                fn(pl.ds(3072 + (sid - 4) * 640, 640))

        @pl.when(cid == 0)
        def _():
            pltpu.sync_copy(init0.at[rng], acc_sp.at[rng])

        @pl.when(cid == 1)
        def _():
            pltpu.sync_copy(init1.at[rng], acc_sp.at[rng])

        pltpu.sync_copy(srcp.at[wid], idx_s)
        pltpu.sync_copy(dstp.at[wid], idx_d)
        if with_cnt:
            cnt_ranges(lambda r: pltpu.sync_copy(zvec.at[r], cnt_sp.at[r]))
            pltpu.sync_copy(ones_hbm, ones_v)
        plsc.subcore_barrier()

        def chunk(j, carry):
            pltpu.async_copy(feat.at[idx_s.at[j]], rows, sem).wait()
            pltpu.sync_copy(rows, acc_sp.at[idx_d.at[j]], add=True)
            if with_cnt:
                pltpu.sync_copy(ones_v, cnt_sp.at[idx_d.at[j]], add=True)
            return carry

        lax.fori_loop(0, CH, chunk, 0)
        plsc.subcore_barrier()
        pltpu.sync_copy(acc_sp.at[rng], agg_out.at[cid].at[rng])
        if with_cnt:
            cnt_ranges(lambda r: pltpu.sync_copy(cnt_sp.at[r],
                                                 cnt_out.at[cid].at[r]))
    return body


def _sc_pass(feat, init0, init1, srcp, dstp, zvec=None, ones=None):
    with_cnt = zvec is not None
    mesh = plsc.VectorSubcoreMesh(core_axis_name="c", subcore_axis_name="s")
    outs = [jax.ShapeDtypeStruct((2, NP, DIM), jnp.float32)]
    scratch = [
        pltpu.VMEM((CH, 128), jnp.int32),
        pltpu.VMEM((CH, 128), jnp.int32),
        pltpu.VMEM((128, DIM), jnp.float32),
    ]
    if with_cnt:
        outs.append(jax.ShapeDtypeStruct((2, NP), jnp.float32))
        scratch.append(pltpu.VMEM((128,), jnp.float32))
    scratch.append(pltpu.VMEM_SHARED((NP, DIM), jnp.float32))
    if with_cnt:
        scratch.append(pltpu.VMEM_SHARED((NP,), jnp.float32))
    scratch.append(pltpu.SemaphoreType.DMA)

    k = pl.kernel(
        _sc_body_factory(with_cnt),
        mesh=mesh,
        out_type=outs if with_cnt else outs[0],
        scratch_types=scratch,
    )
    if with_cnt:
        return k(feat, init0, init1, zvec, ones, srcp, dstp)
    return k(feat, init0, init1, srcp, dstp)


def _full(shape):
    return pl.BlockSpec(shape, lambda i: tuple(0 for _ in shape))


def _rows(shape):
    if len(shape) == 2:
        return pl.BlockSpec(shape, lambda i: (i, 0))
    return pl.BlockSpec(shape, lambda i: (0, i, 0))


def _stage_a(xp, W, b):
    def body(x_ref, w_ref, b_ref, o_ref):
        o_ref[...] = jnp.maximum(
            jnp.dot(x_ref[...], w_ref[...],
                    preferred_element_type=jnp.float32) + b_ref[...], 0.0)
    return pl.pallas_call(
        body,
        grid=(8,),
        in_specs=[_rows((RB, DIM)), _full((DIM, DIM)), _full((1, DIM))],
        out_specs=_rows((RB, DIM)),
        out_shape=jax.ShapeDtypeStruct((NP, DIM), jnp.float32),
    )(xp, W, b)


def _prep(cntp):
    def body(c_ref, rc_ref, di_ref):
        c = c_ref[0] + c_ref[1]
        rc_ref[...] = 1.0 / jnp.maximum(c, 1.0)
        di_ref[...] = lax.rsqrt(c + 1.0)
    return pl.pallas_call(
        body,
        grid=(1,),
        in_specs=[_full((2, NP, 1))],
        out_specs=[_full((NP, 1)), _full((NP, 1))],
        out_shape=[jax.ShapeDtypeStruct((NP, 1), jnp.float32),
                   jax.ShapeDtypeStruct((NP, 1), jnp.float32)],
    )(cntp)


def _stage_b(aggp, h, rc, W_l, b_l, W_r):
    def body(a_ref, h_ref, rc_ref, wl_ref, bl_ref, wr_ref, o_ref):
        mean = (a_ref[0] + a_ref[1]) * rc_ref[...]
        o_ref[...] = jnp.maximum(
            jnp.dot(mean, wl_ref[...], preferred_element_type=jnp.float32)
            + bl_ref[...]
            + jnp.dot(h_ref[...], wr_ref[...],
                      preferred_element_type=jnp.float32), 0.0)
    return pl.pallas_call(
        body,
        grid=(8,),
        in_specs=[_rows((2, RB, DIM)), _rows((RB, DIM)), _rows((RB, 1)),
                  _full((DIM, DIM)), _full((1, DIM)), _full((DIM, DIM))],
        out_specs=_rows((RB, DIM)),
        out_shape=jax.ShapeDtypeStruct((NP, DIM), jnp.float32),
    )(aggp, h, rc, W_l, b_l, W_r)


def _stage_c(aggp, h, rc, dinv, W_l, b_l, W_r, lin1W, lin1b, Wk, Wv):
    def body(a_ref, h_ref, rc_ref, di_ref, wl_ref, bl_ref, wr_ref,
             l1w_ref, l1b_ref, wk_ref, wv_ref, hk_ref, hv_ref):
        mean = (a_ref[0] + a_ref[1]) * rc_ref[...]
        h3 = jnp.maximum(
            jnp.dot(mean, wl_ref[...], preferred_element_type=jnp.float32)
            + bl_ref[...]
            + jnp.dot(h_ref[...], wr_ref[...],
                      preferred_element_type=jnp.float32), 0.0)
        xpv = jnp.dot(h3, l1w_ref[...],
                      preferred_element_type=jnp.float32) + l1b_ref[...]
        di = di_ref[...]
        hk_ref[...] = jnp.dot(xpv, wk_ref[...],
                              preferred_element_type=jnp.float32) * di
        hv_ref[...] = jnp.dot(xpv, wv_ref[...],
                              preferred_element_type=jnp.float32) * di
    return pl.pallas_call(
        body,
        grid=(8,),
        in_specs=[_rows((2, RB, DIM)), _rows((RB, DIM)), _rows((RB, 1)),
                  _rows((RB, 1)), _full((DIM, DIM)), _full((1, DIM)),
                  _full((DIM, DIM)), _full((DIM, DIM)), _full((1, DIM)),
                  _full((DIM, DIM)), _full((DIM, DIM))],
        out_specs=[_rows((RB, DIM)), _rows((RB, DIM))],
        out_shape=[jax.ShapeDtypeStruct((NP, DIM), jnp.float32),
                   jax.ShapeDtypeStruct((NP, DIM), jnp.float32)],
    )(aggp, h, rc, dinv, W_l, b_l, W_r, lin1W, lin1b, Wk, Wv)


def _softmax_rows(s):
    m = jnp.max(s, axis=1, keepdims=True)
    e = jnp.exp(s - m)
    return e / jnp.sum(e, axis=1, keepdims=True)


def _heads_attn(Qp, K, V, maskrow=None):
    outs = []
    for h in range(4):
        qh = Qp[:, h * 32:(h + 1) * 32]
        kh = K[:, h * 32:(h + 1) * 32]
        vh = V[:, h * 32:(h + 1) * 32]
        s = lax.dot_general(qh, kh, (((1,), (1,)), ((), ())),
                            preferred_element_type=jnp.float32) * ISQ
        if maskrow is not None:
            s = s + maskrow
        a = _softmax_rows(s)
        outs.append(jnp.dot(a, vh, preferred_element_type=jnp.float32))
    return Qp + jnp.concatenate(outs, axis=1)


def _attention(kn0, kn1, vn0, vn1, dinv, batch2d, wts):
    (S1, Wq1, bq1, Wo1, bo1, bk1, bv1,
     Wq2, bq2, Wk2, bk2, Wv2, bv2, Wo2, bo2,
     S3, Wq3, bq3, Wk3, bk3, Wv3, bv3, Wo3, bo3,
     l2W, l2b) = wts

    def body(k0_ref, k1_ref, v0_ref, v1_ref, di_ref, b_ref,
             s1_ref, wq1_ref, bq1_ref, wo1_ref, bo1_ref, bk1_ref, bv1_ref,
             wq2_ref, bq2_ref, wk2_ref, bk2_ref, wv2_ref, bv2_ref,
             wo2_ref, bo2_ref,
             s3_ref, wq3_ref, bq3_ref, wk3_ref, bk3_ref, wv3_ref, bv3_ref,
             wo3_ref, bo3_ref, l2w_ref, l2b_ref, o_ref):
        g = pl.program_id(0)
        b2 = b_ref[...]
        cntg = jnp.sum((b2 == g).astype(jnp.int32))
        startg = jnp.sum((b2 < g).astype(jnp.int32))
        sl = pl.ds(startg, 640)
        di = di_ref[sl, :]
        K = di * (k0_ref[sl, :] + k1_ref[sl, :]) + bk1_ref[...]
        V = di * (v0_ref[sl, :] + v1_ref[sl, :]) + bv1_ref[...]
        kmax = jnp.minimum(cntg, MAXN)
        vcol = lax.broadcasted_iota(jnp.int32, (640, 1), 0) < kmax
        vrow = lax.broadcasted_iota(jnp.int32, (1, 640), 1) < kmax
        V = jnp.where(vcol, V, 0.0)
        maskrow = jnp.where(vrow, 0.0, NEG)

        Qp = jnp.dot(s1_ref[...], wq1_ref[...],
                     preferred_element_type=jnp.float32) + bq1_ref[...]
        X = _heads_attn(Qp, K, V, maskrow)
        X = X + jnp.maximum(
            jnp.dot(X, wo1_ref[...], preferred_element_type=jnp.float32)
            + bo1_ref[...], 0.0)

        Q2 = jnp.dot(X, wq2_ref[...],
                     preferred_element_type=jnp.float32) + bq2_ref[...]
        K2 = jnp.dot(X, wk2_ref[...],
                     preferred_element_type=jnp.float32) + bk2_ref[...]
        V2 = jnp.dot(X, wv2_ref[...],
                     preferred_element_type=jnp.float32) + bv2_ref[...]
        X2 = _heads_attn(Q2, K2, V2)
        X2 = X2 + jnp.maximum(
            jnp.dot(X2, wo2_ref[...], preferred_element_type=jnp.float32)
            + bo2_ref[...], 0.0)

        Q3 = jnp.dot(s3_ref[...], wq3_ref[...],
                     preferred_element_type=jnp.float32) + bq3_ref[...]
        K3 = jnp.dot(X2, wk3_ref[...],
                     preferred_element_type=jnp.float32) + bk3_ref[...]
        V3 = jnp.dot(X2, wv3_ref[...],
                     preferred_element_type=jnp.float32) + bv3_ref[...]
        X3 = _heads_attn(Q3, K3, V3)
        X3 = X3 + jnp.maximum(
            jnp.dot(X3, wo3_ref[...], preferred_element_type=jnp.float32)
            + bo3_ref[...], 0.0)
        y = jnp.dot(X3, l2w_ref[...],
                    preferred_element_type=jnp.float32) + l2b_ref[...]
        o_ref[...] = jnp.broadcast_to(y[None], (1, 8, DIM))

    nspecs = [_full((NP, DIM))] * 4 + [_full((NP, 1)), _full((84, 128))]
    wspecs = [_full(w.shape) for w in wts]
    return pl.pallas_call(
        body,
        grid=(NG,),
        in_specs=nspecs + wspecs,
        out_specs=pl.BlockSpec((1, 8, DIM), lambda i: (i, 0, 0)),
        out_shape=jax.ShapeDtypeStruct((NG, 8, DIM), jnp.float32),
    )(kn0, kn1, vn0, vn1, dinv, batch2d, *wts)


def kernel(x, edge_index, batch, params):
    p = params
    xpad = jnp.pad(x, ((0, NP - N), (0, 0)))
    srcp = jnp.pad(edge_index[0], (0, EP - E),
                   constant_values=N).reshape(NW, CH, 128)
    dstp = jnp.pad(edge_index[1], (0, EP - E),
                   constant_values=N).reshape(NW, CH, 128)
    batch2d = jnp.pad(batch, (0, NP - N),
                      constant_values=NG).reshape(84, 128)
    z128 = jnp.zeros((NP, DIM), jnp.float32)
    zvec = jnp.zeros((NP,), jnp.float32)
    ones = jnp.ones((128,), jnp.float32)

    def r2(b):
        return b.reshape(1, DIM)

    h0 = _stage_a(xpad, p["W_in"], r2(p["b_in"]))
    agg0, cntp = _sc_pass(h0, z128, z128, srcp, dstp, zvec=zvec, ones=ones)
    rc, dinv = _prep(cntp.reshape(2, NP, 1))
    h1 = _stage_b(agg0, h0, rc, p["W_l0"], r2(p["b_l0"]), p["W_r0"])
    agg1 = _sc_pass(h1, z128, z128, srcp, dstp)
    h2 = _stage_b(agg1, h1, rc, p["W_l1"], r2(p["b_l1"]), p["W_r1"])
    agg2 = _sc_pass(h2, z128, z128, srcp, dstp)
    hk, hv = _stage_c(agg2, h2, rc, dinv, p["W_l2"], r2(p["b_l2"]),
                      p["W_r2"], p["lin1_W"], r2(p["lin1_b"]),
                      p["p1"]["Wk"], p["p1"]["Wv"])
    kp = _sc_pass(hk, hk, z128, srcp, dstp)
    vp = _sc_pass(hv, hv, z128, srcp, dstp)

    m1, m2, m3 = p["p1"], p["p2"], p["p3"]
    wts = (p["S1"][0], m1["Wq"], r2(m1["bq"]), m1["Wo"], r2(m1["bo"]),
           r2(m1["bk"]), r2(m1["bv"]),
           m2["Wq"], r2(m2["bq"]), m2["Wk"], r2(m2["bk"]),
           m2["Wv"], r2(m2["bv"]), m2["Wo"], r2(m2["bo"]),
           p["S3"][0], m3["Wq"], r2(m3["bq"]), m3["Wk"], r2(m3["bk"]),
           m3["Wv"], r2(m3["bv"]), m3["Wo"], r2(m3["bo"]),
           p["lin2_W"], p["lin2_b"].reshape(1, 1))
    y = _attention(kp[0], kp[1], vp[0], vp[1], dinv, batch2d, wts)
    return y[:, 0, 0]
